# Initial kernel scaffold; baseline (speedup 1.0000x reference)
import functools

import jax
import jax.numpy as jnp
from jax import lax
from jax.experimental import pallas as pl
from jax.experimental.pallas import tpu as pltpu
from jax.experimental.pallas import tpu_sc as plsc

N = 10000
E = 320000
D = 128
NC = 2
NS = 16
NW = NC * NS
L = 16
CH = 80
NCH = (E // NW) // CH
NPT = N // NS
DEGW = 16
ROWS2D = E // CH


def _ew_tc(ec_ref, ew_ref):
    e = ec_ref[...]
    mn = jnp.min(e)
    mx = jnp.max(e)
    ew_ref[...] = (e - mn) / (mx - mn)


def _edge_weights(ec2d):
    return pl.pallas_call(
        _ew_tc,
        out_shape=jax.ShapeDtypeStruct(ec2d.shape, jnp.float32),
    )(ec2d)


def _mm_tc(x_ref, w_ref, o_ref):
    o_ref[...] = jnp.dot(x_ref[...], w_ref[...],
                         preferred_element_type=jnp.float32)


def _matmul(x, w, bn=2000):
    n = x.shape[0]
    return pl.pallas_call(
        _mm_tc,
        grid=(n // bn,),
        in_specs=[
            pl.BlockSpec((bn, D), lambda i: (i, 0)),
            pl.BlockSpec((D, D), lambda i: (0, 0)),
        ],
        out_specs=pl.BlockSpec((bn, D), lambda i: (i, 0)),
        out_shape=jax.ShapeDtypeStruct((n, D), jnp.float32),
    )(x, w)


def _dis_tc(degp_ref, dis_ref):
    d = degp_ref[0, :, :1] + degp_ref[1, :, :1] + 1.0
    dis_ref[...] = jax.lax.rsqrt(jnp.maximum(d, 1e-12))


def _deg_inv_sqrt(degp):
    return pl.pallas_call(
        _dis_tc,
        out_shape=jax.ShapeDtypeStruct((N, 1), jnp.float32),
    )(degp)


def _post1_tc(aggp_ref, xw_ref, dis_ref, b_ref, w2_ref, o_ref):
    dis = dis_ref[...]
    h = aggp_ref[0] + aggp_ref[1] + dis * dis * xw_ref[...] + b_ref[...]
    h = jnp.maximum(h, 0.0)
    o_ref[...] = jnp.dot(h, w2_ref[...], preferred_element_type=jnp.float32)


def _layer1_post(aggp, xw1, dis, b1, w2, bn=2000):
    return pl.pallas_call(
        _post1_tc,
        grid=(N // bn,),
        in_specs=[
            pl.BlockSpec((2, bn, D), lambda i: (0, i, 0)),
            pl.BlockSpec((bn, D), lambda i: (i, 0)),
            pl.BlockSpec((bn, 1), lambda i: (i, 0)),
            pl.BlockSpec((1, D), lambda i: (0, 0)),
            pl.BlockSpec((D, D), lambda i: (0, 0)),
        ],
        out_specs=pl.BlockSpec((bn, D), lambda i: (i, 0)),
        out_shape=jax.ShapeDtypeStruct((N, D), jnp.float32),
    )(aggp, xw1, dis, b1, w2)


def _post2_tc(aggp_ref, xw_ref, dis_ref, b_ref, h_ref, ls_ref):
    dis = dis_ref[...]
    h = aggp_ref[0] + aggp_ref[1] + dis * dis * xw_ref[...] + b_ref[...]
    h_ref[...] = h
    m = jnp.max(h, axis=-1, keepdims=True)
    lse = jnp.log(jnp.sum(jnp.exp(h - m), axis=-1, keepdims=True)) + m
    ls_ref[...] = h - lse


def _layer2_post(aggp, xw2, dis, b2, bn=2000):
    return pl.pallas_call(
        _post2_tc,
        grid=(N // bn,),
        in_specs=[
            pl.BlockSpec((2, bn, D), lambda i: (0, i, 0)),
            pl.BlockSpec((bn, D), lambda i: (i, 0)),
            pl.BlockSpec((bn, 1), lambda i: (i, 0)),
            pl.BlockSpec((1, D), lambda i: (0, 0)),
        ],
        out_specs=[
            pl.BlockSpec((bn, D), lambda i: (i, 0)),
            pl.BlockSpec((bn, D), lambda i: (i, 0)),
        ],
        out_shape=[
            jax.ShapeDtypeStruct((N, D), jnp.float32),
            jax.ShapeDtypeStruct((N, D), jnp.float32),
        ],
    )(aggp, xw2, dis, b2)


_MESH = dict(core_axis_name="c", subcore_axis_name="s")


def _degree_partials(col2d, ew2d):
    @functools.partial(
        pl.kernel,
        out_type=jax.ShapeDtypeStruct((NC, N, DEGW), jnp.float32),
        mesh=plsc.VectorSubcoreMesh(**_MESH),
        scratch_types=[
            pltpu.VMEM((NCH, CH), jnp.int32),
            pltpu.VMEM((NCH, CH), jnp.float32),
            pltpu.VMEM((CH, DEGW), jnp.float32),
            pltpu.VMEM((NPT, DEGW), jnp.float32),
            pltpu.VMEM_SHARED((N, DEGW), jnp.float32),
        ],
    )
    def deg_kernel(col_hbm, ew_hbm, out_hbm, col_v, ew_v, msg_v, z_v, degw):
        c = lax.axis_index("c")
        s = lax.axis_index("s")
        w = s * NC + c

        def zrow(j, carry):
            z_v[j, :] = jnp.zeros((DEGW,), jnp.float32)
            return carry

        lax.fori_loop(0, NPT, zrow, 0)
        pltpu.sync_copy(z_v, degw.at[pl.ds(s * NPT, NPT)])
        plsc.subcore_barrier()

        pltpu.sync_copy(col_hbm.at[pl.ds(w * NCH, NCH)], col_v)
        pltpu.sync_copy(ew_hbm.at[pl.ds(w * NCH, NCH)], ew_v)

        def chunk(j, carry):
            def edge(j2, c2):
                msg_v[j2, :] = jnp.full((DEGW,), ew_v[j, j2], jnp.float32)
                return c2

            lax.fori_loop(0, CH, edge, 0)
            pltpu.sync_copy(msg_v, degw.at[col_v.at[j]], add=True)
            return carry

        lax.fori_loop(0, NCH, chunk, 0)
        plsc.subcore_barrier()
        pltpu.sync_copy(degw.at[pl.ds(s * NPT, NPT)],
                        out_hbm.at[c, pl.ds(s * NPT, NPT)])

    return deg_kernel(col2d, ew2d)


def _spmm_body(compute_coef, xw_hbm, row_hbm, col_hbm, sc_hbm, dis_hbm,
               aggp_hbm, coef_hbm, row_v, col_v, sc_v, coef_v, dis_v,
               msgs_v, acc, sem):
    c = lax.axis_index("c")
    s = lax.axis_index("s")
    w = s * NC + c

    def zrow(j, carry):
        for u in range(D // L):
            msgs_v[j, pl.ds(u * L, L)] = jnp.zeros((L,), jnp.float32)
        return carry

    lax.fori_loop(0, CH, zrow, 0)
    nfull = NPT // CH
    for q in range(nfull):
        pltpu.sync_copy(msgs_v, acc.at[pl.ds(s * NPT + q * CH, CH)])
    rem = NPT - nfull * CH
    if rem:
        pltpu.sync_copy(msgs_v.at[pl.ds(0, rem)],
                        acc.at[pl.ds(s * NPT + nfull * CH, rem)])

    pltpu.sync_copy(row_hbm.at[pl.ds(w * NCH, NCH)], row_v)
    pltpu.sync_copy(col_hbm.at[pl.ds(w * NCH, NCH)], col_v)
    pltpu.sync_copy(sc_hbm.at[pl.ds(w * NCH, NCH)], sc_v)

    if compute_coef:
        pltpu.sync_copy(dis_hbm, dis_v)

        def cchunk(j, carry):
            for t in range(CH // L):
                rv = row_v[j, pl.ds(t * L, L)]
                cv = col_v[j, pl.ds(t * L, L)]
                ev = sc_v[j, pl.ds(t * L, L)]
                dr = plsc.load_gather(dis_v, [rv])
                dc = plsc.load_gather(dis_v, [cv])
                coef_v[j, pl.ds(t * L, L)] = dr * ev * dc
            return carry

        lax.fori_loop(0, NCH, cchunk, 0)
        pltpu.sync_copy(coef_v, coef_hbm.at[pl.ds(w * NCH, NCH)])
        use_v = coef_v
    else:
        use_v = sc_v

    plsc.subcore_barrier()

    def chunk(j, carry):
        pltpu.async_copy(xw_hbm.at[row_v.at[j]], msgs_v, sem).wait()

        def edge(j2, c2):
            sval = use_v[j, j2]
            for u in range(D // L):
                msgs_v[j2, pl.ds(u * L, L)] = \
                    msgs_v[j2, pl.ds(u * L, L)] * sval
            return c2

        lax.fori_loop(0, CH, edge, 0)
        pltpu.sync_copy(msgs_v, acc.at[col_v.at[j]], add=True)
        return carry

    lax.fori_loop(0, NCH, chunk, 0)
    plsc.subcore_barrier()
    pltpu.sync_copy(acc.at[pl.ds(s * NPT, NPT)],
                    aggp_hbm.at[c, pl.ds(s * NPT, NPT)])


_SPMM_SCRATCH = [
    pltpu.VMEM((NCH, CH), jnp.int32),
    pltpu.VMEM((NCH, CH), jnp.int32),
    pltpu.VMEM((NCH, CH), jnp.float32),
    pltpu.VMEM((NCH, CH), jnp.float32),
    pltpu.VMEM((N,), jnp.float32),
    pltpu.VMEM((CH, D), jnp.float32),
    pltpu.VMEM_SHARED((N, D), jnp.float32),
    pltpu.SemaphoreType.DMA,
]


def _spmm_layer1(xw, row2d, col2d, ew2d, dis_flat):
    @functools.partial(
        pl.kernel,
        out_type=(jax.ShapeDtypeStruct((NC, N, D), jnp.float32),
                  jax.ShapeDtypeStruct((ROWS2D, CH), jnp.float32)),
        mesh=plsc.VectorSubcoreMesh(**_MESH),
        scratch_types=_SPMM_SCRATCH,
    )
    def k(xw_hbm, row_hbm, col_hbm, ew_hbm, dis_hbm, aggp_hbm, coef_hbm,
          *scratch):
        _spmm_body(True, xw_hbm, row_hbm, col_hbm, ew_hbm, dis_hbm,
                   aggp_hbm, coef_hbm, *scratch)

    return k(xw, row2d, col2d, ew2d, dis_flat)


def _spmm_layer2(xw, row2d, col2d, coef2d):
    @functools.partial(
        pl.kernel,
        out_type=jax.ShapeDtypeStruct((NC, N, D), jnp.float32),
        mesh=plsc.VectorSubcoreMesh(**_MESH),
        scratch_types=_SPMM_SCRATCH,
    )
    def k(xw_hbm, row_hbm, col_hbm, coef_hbm, aggp_hbm, *scratch):
        _spmm_body(False, xw_hbm, row_hbm, col_hbm, coef_hbm, None,
                   aggp_hbm, None, *scratch)

    return k(xw, row2d, col2d, coef2d)


def kernel(x, edge_index, edge_count, W1, b1, W2, b2):
    row2d = edge_index[0].astype(jnp.int32).reshape(ROWS2D, CH)
    col2d = edge_index[1].astype(jnp.int32).reshape(ROWS2D, CH)
    ec2d = edge_count[:, 0].reshape(E // D, D)

    ew2d = _edge_weights(ec2d).reshape(ROWS2D, CH)
    degp = _degree_partials(col2d, ew2d)
    dis = _deg_inv_sqrt(degp)

    xw1 = _matmul(x, W1)
    aggp1, coef2d = _spmm_layer1(xw1, row2d, col2d, ew2d, dis.reshape(N))
    xw2 = _layer1_post(aggp1, xw1, dis, b1.reshape(1, D), W2)
    aggp2 = _spmm_layer2(xw2, row2d, col2d, coef2d)
    h2, ls = _layer2_post(aggp2, xw2, dis, b2.reshape(1, D))
    return (h2, ls)


# speedup vs baseline: 16.7446x; 16.7446x over previous
import functools

import jax
import jax.numpy as jnp
from jax import lax
from jax.experimental import pallas as pl
from jax.experimental.pallas import tpu as pltpu
from jax.experimental.pallas import tpu_sc as plsc

N = 10000
E = 320000
D = 128
NC = 2
NS = 16
NW = NC * NS
L = 16
CH = 80
NCH = (E // NW) // CH
EW = E // NW
NP = 10240
NPT = NP // NS
GRP = CH // L


def _ew_tc(ec_ref, ew_ref):
    e = ec_ref[...]
    mn = jnp.min(e)
    mx = jnp.max(e)
    ew_ref[...] = (e - mn) / (mx - mn)


def _edge_weights(ec2d):
    return pl.pallas_call(
        _ew_tc,
        out_shape=jax.ShapeDtypeStruct(ec2d.shape, jnp.float32),
    )(ec2d)


def _dis_tc(degp_ref, dis_ref):
    d = degp_ref[:1, :N] + degp_ref[1:, :N] + 1.0
    dis_ref[...] = jax.lax.rsqrt(jnp.maximum(d, 1e-12))


def _deg_inv_sqrt(degp2d):
    return pl.pallas_call(
        _dis_tc,
        out_shape=jax.ShapeDtypeStruct((1, N), jnp.float32),
    )(degp2d)


def _mm_tc(x_ref, w_ref, dis_ref, o_ref):
    xw = jnp.dot(x_ref[...], w_ref[...], preferred_element_type=jnp.float32)
    o_ref[...] = dis_ref[...] * xw


def _matmul_prescaled(x, w, dis, bn=2000):
    return pl.pallas_call(
        _mm_tc,
        grid=(N // bn,),
        in_specs=[
            pl.BlockSpec((bn, D), lambda i: (i, 0)),
            pl.BlockSpec((D, D), lambda i: (0, 0)),
            pl.BlockSpec((bn, 1), lambda i: (i, 0)),
        ],
        out_specs=pl.BlockSpec((bn, D), lambda i: (i, 0)),
        out_shape=jax.ShapeDtypeStruct((N, D), jnp.float32),
    )(x, w, dis)


def _post1_tc(aggp_ref, yw_ref, dis_ref, b_ref, w2_ref, o_ref):
    dis = dis_ref[...]
    h = dis * (aggp_ref[0] + aggp_ref[1] + yw_ref[...]) + b_ref[...]
    h = jnp.maximum(h, 0.0)
    o_ref[...] = dis * jnp.dot(h, w2_ref[...],
                               preferred_element_type=jnp.float32)


def _layer1_post(aggp, yw1, dis, b1, w2, bn=2000):
    return pl.pallas_call(
        _post1_tc,
        grid=(N // bn,),
        in_specs=[
            pl.BlockSpec((2, bn, D), lambda i: (0, i, 0)),
            pl.BlockSpec((bn, D), lambda i: (i, 0)),
            pl.BlockSpec((bn, 1), lambda i: (i, 0)),
            pl.BlockSpec((1, D), lambda i: (0, 0)),
            pl.BlockSpec((D, D), lambda i: (0, 0)),
        ],
        out_specs=pl.BlockSpec((bn, D), lambda i: (i, 0)),
        out_shape=jax.ShapeDtypeStruct((N, D), jnp.float32),
    )(aggp, yw1, dis, b1, w2)


def _post2_tc(aggp_ref, yw_ref, dis_ref, b_ref, h_ref, ls_ref):
    dis = dis_ref[...]
    h = dis * (aggp_ref[0] + aggp_ref[1] + yw_ref[...]) + b_ref[...]
    h_ref[...] = h
    m = jnp.max(h, axis=-1, keepdims=True)
    lse = jnp.log(jnp.sum(jnp.exp(h - m), axis=-1, keepdims=True)) + m
    ls_ref[...] = h - lse


def _layer2_post(aggp, yw2, dis, b2, bn=2000):
    return pl.pallas_call(
        _post2_tc,
        grid=(N // bn,),
        in_specs=[
            pl.BlockSpec((2, bn, D), lambda i: (0, i, 0)),
            pl.BlockSpec((bn, D), lambda i: (i, 0)),
            pl.BlockSpec((bn, 1), lambda i: (i, 0)),
            pl.BlockSpec((1, D), lambda i: (0, 0)),
        ],
        out_specs=[
            pl.BlockSpec((bn, D), lambda i: (i, 0)),
            pl.BlockSpec((bn, D), lambda i: (i, 0)),
        ],
        out_shape=[
            jax.ShapeDtypeStruct((N, D), jnp.float32),
            jax.ShapeDtypeStruct((N, D), jnp.float32),
        ],
    )(aggp, yw2, dis, b2)


_MESH = dict(core_axis_name="c", subcore_axis_name="s")


@functools.partial(
    pl.kernel,
    out_type=jax.ShapeDtypeStruct((NC, NS, 1, NPT), jnp.float32),
    mesh=plsc.VectorSubcoreMesh(**_MESH),
    scratch_types=[
        pltpu.VMEM((NCH, CH), jnp.int32),
        pltpu.VMEM((EW,), jnp.float32),
        pltpu.VMEM((NPT,), jnp.float32),
        pltpu.VMEM_SHARED((NP,), jnp.float32),
    ],
)
def _deg_kernel(col_hbm, ew_hbm, out_hbm, col_v, ew_v, z_v, degw):
    c = lax.axis_index("c")
    s = lax.axis_index("s")
    w = s * NC + c

    def zgrp(j, carry):
        z_v[pl.ds(j * L, L)] = jnp.zeros((L,), jnp.float32)
        return carry

    lax.fori_loop(0, NPT // L, zgrp, 0)
    pltpu.sync_copy(z_v, degw.at[pl.ds(s * NPT, NPT)])
    plsc.subcore_barrier()

    pltpu.sync_copy(col_hbm.at[w], col_v)
    pltpu.sync_copy(ew_hbm.at[pl.ds(w * EW, EW)], ew_v)

    def chunk(j, carry):
        pltpu.sync_copy(ew_v.at[pl.ds(j * CH, CH)],
                        degw.at[col_v.at[j]], add=True)
        return carry

    lax.fori_loop(0, NCH, chunk, 0)
    plsc.subcore_barrier()
    pltpu.sync_copy(degw.at[pl.ds(s * NPT, NPT)], out_hbm.at[c, s, 0])


@functools.partial(
    pl.kernel,
    out_type=jax.ShapeDtypeStruct((NC, NP, D), jnp.float32),
    mesh=plsc.VectorSubcoreMesh(**_MESH),
    scratch_types=[
        pltpu.VMEM((EW,), jnp.int32),
        pltpu.VMEM((NCH, CH), jnp.int32),
        pltpu.VMEM((EW,), jnp.float32),
        pltpu.VMEM((CH, D), jnp.float32),
        pltpu.VMEM_SHARED((NP, D), jnp.float32),
        pltpu.SemaphoreType.DMA,
    ],
)
def _spmm_kernel(yw_hbm, rowf_hbm, col_hbm, ewf_hbm, aggp_hbm,
                 rowf_v, col_v, ewf_v, msgs_v, acc, sem):
    c = lax.axis_index("c")
    s = lax.axis_index("s")
    w = s * NC + c

    def zrow(j, carry):
        for u in range(D // L):
            msgs_v[j, pl.ds(u * L, L)] = jnp.zeros((L,), jnp.float32)
        return carry

    lax.fori_loop(0, CH, zrow, 0)
    for q in range(NPT // CH):
        pltpu.sync_copy(msgs_v, acc.at[pl.ds(s * NPT + q * CH, CH)])

    pltpu.sync_copy(rowf_hbm.at[pl.ds(w * EW, EW)], rowf_v)
    pltpu.sync_copy(col_hbm.at[w], col_v)
    pltpu.sync_copy(ewf_hbm.at[pl.ds(w * EW, EW)], ewf_v)
    plsc.subcore_barrier()

    def chunk(j, carry):
        pltpu.async_copy(
            yw_hbm.at[rowf_v.at[pl.ds(j * CH, CH)]], msgs_v, sem).wait()
        for g in range(GRP):
            evec = ewf_v[pl.ds(j * CH + g * L, L)]
            for l in range(L):
                svec = jnp.full((L,), evec[l], jnp.float32)
                for u in range(D // L):
                    msgs_v[g * L + l, pl.ds(u * L, L)] = \
                        msgs_v[g * L + l, pl.ds(u * L, L)] * svec
        pltpu.sync_copy(msgs_v, acc.at[col_v.at[j]], add=True)
        return carry

    lax.fori_loop(0, NCH, chunk, 0)
    plsc.subcore_barrier()
    pltpu.sync_copy(acc.at[pl.ds(s * NPT, NPT)],
                    aggp_hbm.at[c, pl.ds(s * NPT, NPT)])


def kernel(x, edge_index, edge_count, W1, b1, W2, b2):
    rowf = edge_index[0].astype(jnp.int32)
    col3d = edge_index[1].astype(jnp.int32).reshape(NW, NCH, CH)
    ec2d = edge_count[:, 0].reshape(E // D, D)

    ewf = _edge_weights(ec2d).reshape(E)
    degp = _deg_kernel(col3d, ewf)
    dis = _deg_inv_sqrt(degp.reshape(NC, NP))
    dis = dis.reshape(N, 1)

    yw1 = _matmul_prescaled(x, W1, dis)
    aggp1 = _spmm_kernel(yw1, rowf, col3d, ewf)
    yw2 = _layer1_post(aggp1, yw1, dis, b1.reshape(1, D), W2)
    aggp2 = _spmm_kernel(yw2, rowf, col3d, ewf)
    h2, ls = _layer2_post(aggp2, yw2, dis, b2.reshape(1, D))
    return (h2, ls)
